# Initial kernel scaffold; baseline (speedup 1.0000x reference)
import jax
import jax.numpy as jnp
from jax.experimental import pallas as pl


def kernel(PE, i):
    raise NotImplementedError("write your pallas kernel here")


# speedup vs baseline: 6.3516x; 6.3516x over previous
import functools

import jax
import jax.numpy as jnp
from jax import lax
from jax.experimental import pallas as pl
from jax.experimental.pallas import tpu as pltpu
from jax.experimental.pallas import tpu_sc as plsc

HID = 128
NC = 2
NS = 16
NW = NC * NS
CH = 128


def _make_gather(n_total):
    n_per_w = n_total // NW
    nch = n_per_w // CH
    mesh = plsc.VectorSubcoreMesh(core_axis_name="c", subcore_axis_name="s")

    @functools.partial(
        pl.kernel,
        mesh=mesh,
        out_type=jax.ShapeDtypeStruct((NW, nch, CH, HID), jnp.float32),
        scratch_types=[
            pltpu.VMEM((nch, CH), jnp.int32),
            pltpu.VMEM((CH, HID), jnp.float32),
            pltpu.SemaphoreType.DMA,
        ],
    )
    def k(table_hbm, idx_hbm, out_hbm, idx_v, rows_v, sem):
        wid = lax.axis_index("s") * NC + lax.axis_index("c")
        pltpu.sync_copy(idx_hbm.at[wid], idx_v)

        def body(j, carry):
            pltpu.async_copy(table_hbm.at[idx_v.at[j]], rows_v, sem).wait()
            pltpu.sync_copy(rows_v, out_hbm.at[wid, j])
            return carry

        lax.fori_loop(0, nch, body, 0)

    return k


def kernel(PE, i):
    n_total = i.shape[0] * i.shape[1]
    idx = i.reshape(NW, n_total // (NW * CH), CH)
    out = _make_gather(n_total)(PE, idx)
    return out.reshape(i.shape[0], i.shape[1], HID)


# speedup vs baseline: 9.1951x; 1.4477x over previous
import functools

import jax
import jax.numpy as jnp
from jax import lax
from jax.experimental import pallas as pl
from jax.experimental.pallas import tpu as pltpu
from jax.experimental.pallas import tpu_sc as plsc

HID = 128
NC = 2
NS = 16
NW = NC * NS
CH = 128
NBUF = 5


def _make_gather(n_total):
    n_per_w = n_total // NW
    nch = n_per_w // CH
    assert nch % NBUF == 0 and nch >= 2 * NBUF
    mesh = plsc.VectorSubcoreMesh(core_axis_name="c", subcore_axis_name="s")

    scratch = [
        pltpu.VMEM((nch, CH), jnp.int32),
        pltpu.VMEM((NBUF, CH, HID), jnp.float32),
    ] + [pltpu.SemaphoreType.DMA] * (2 * NBUF)

    @functools.partial(
        pl.kernel,
        mesh=mesh,
        out_type=jax.ShapeDtypeStruct((NW, nch, CH, HID), jnp.float32),
        scratch_types=scratch,
    )
    def k(table_hbm, idx_hbm, out_hbm, idx_v, rows_v, *sems):
        gsem, osem = sems[:NBUF], sems[NBUF:]
        wid = lax.axis_index("s") * NC + lax.axis_index("c")
        pltpu.sync_copy(idx_hbm.at[wid], idx_v)

        def gather(j, b):
            return pltpu.make_async_copy(
                table_hbm.at[idx_v.at[j]], rows_v.at[b], gsem[b])

        def store(j, b):
            return pltpu.make_async_copy(
                rows_v.at[b], out_hbm.at[wid, j], osem[b])

        def slot(j, b, first=False, last=False):
            gather(j, b).wait()
            store(j, b).start()
            if not last:
                bn = (b + NBUF - 1) % NBUF
                if not first:
                    store(j - 1, bn).wait()
                gather(j + NBUF - 1, bn).start()

        for b in range(NBUF - 1):
            gather(b, b).start()

        slot(0, 0, first=True)
        for b in range(1, NBUF):
            slot(b, b)

        def body(g, carry):
            j0 = g * NBUF
            for b in range(NBUF):
                slot(j0 + b, b)
            return carry

        lax.fori_loop(1, nch // NBUF - 1, body, 0)

        j0 = nch - NBUF
        slot(j0, 0, last=False)
        for b in range(1, NBUF):
            slot(j0 + b, b, last=True)

        for b in range(NBUF):
            store(j0 + b, b).wait()

    return k


def kernel(PE, i):
    n_total = i.shape[0] * i.shape[1]
    idx = i.reshape(NW, n_total // (NW * CH), CH)
    out = _make_gather(n_total)(PE, idx)
    return out.reshape(i.shape[0], i.shape[1], HID)


# speedup vs baseline: 9.2149x; 1.0021x over previous
import functools

import jax
import jax.numpy as jnp
from jax import lax
from jax.experimental import pallas as pl
from jax.experimental.pallas import tpu as pltpu
from jax.experimental.pallas import tpu_sc as plsc

HID = 128
NC = 2
NS = 16
NW = NC * NS
CH = 128
NBUF = 5
LEAD = 3


def _make_gather(n_total):
    n_per_w = n_total // NW
    nch = n_per_w // CH
    assert nch % NBUF == 0 and nch >= 2 * NBUF
    mesh = plsc.VectorSubcoreMesh(core_axis_name="c", subcore_axis_name="s")

    scratch = [
        pltpu.VMEM((nch, CH), jnp.int32),
        pltpu.VMEM((NBUF, CH, HID), jnp.float32),
    ] + [pltpu.SemaphoreType.DMA] * (2 * NBUF)

    @functools.partial(
        pl.kernel,
        mesh=mesh,
        out_type=jax.ShapeDtypeStruct((NW, nch, CH, HID), jnp.float32),
        scratch_types=scratch,
    )
    def k(table_hbm, idx_hbm, out_hbm, idx_v, rows_v, *sems):
        gsem, osem = sems[:NBUF], sems[NBUF:]
        wid = lax.axis_index("s") * NC + lax.axis_index("c")
        pltpu.sync_copy(idx_hbm.at[wid], idx_v)

        def gather(j, b):
            return pltpu.make_async_copy(
                table_hbm.at[idx_v.at[j]], rows_v.at[b], gsem[b])

        def store(j, b):
            return pltpu.make_async_copy(
                rows_v.at[b], out_hbm.at[wid, j], osem[b])

        def slot(j, b, do_owait=True, do_gstart=True):
            gather(j, b).wait()
            store(j, b).start()
            if do_gstart:
                bn = (b + LEAD) % NBUF
                if do_owait:
                    store(j + LEAD - NBUF, bn).wait()
                gather(j + LEAD, bn).start()

        for b in range(LEAD):
            gather(b, b).start()

        for b in range(NBUF):
            slot(b, b, do_owait=(b + LEAD >= NBUF))

        def body(g, carry):
            j0 = g * NBUF
            for b in range(NBUF):
                slot(j0 + b, b)
            return carry

        lax.fori_loop(1, nch // NBUF - 1, body, 0)

        j0 = nch - NBUF
        for b in range(NBUF):
            slot(j0 + b, b, do_gstart=(b + LEAD < NBUF))

        for b in range(NBUF):
            store(j0 + b, b).wait()

    return k


def kernel(PE, i):
    n_total = i.shape[0] * i.shape[1]
    idx = i.reshape(NW, n_total // (NW * CH), CH)
    out = _make_gather(n_total)(PE, idx)
    return out.reshape(i.shape[0], i.shape[1], HID)
